# Initial kernel scaffold; baseline (speedup 1.0000x reference)
import jax
import jax.numpy as jnp
from jax.experimental import pallas as pl


def kernel(input_ids, encoder_embed_scale, decoder_input_ids, decoder_embed_scale, shared_weight):
    raise NotImplementedError("write your pallas kernel here")


# speedup vs baseline: 4.9098x; 4.9098x over previous
import functools

import jax
import jax.numpy as jnp
from jax import lax
from jax.experimental import pallas as pl
from jax.experimental.pallas import tpu as pltpu
from jax.experimental.pallas import tpu_sc as plsc

LANES = 16
CHUNK = 128


def _make_sc_lookup(vocab, d, n_rows):
  info = plsc.get_sparse_core_info()
  nw = info.num_cores * info.num_subcores
  assert n_rows % (nw * CHUNK) == 0
  per_w = n_rows // nw
  nchunks = per_w // CHUNK

  mesh = plsc.VectorSubcoreMesh(core_axis_name="c", subcore_axis_name="s")
  out_sd = jax.ShapeDtypeStruct((n_rows, d), jnp.float32)

  @functools.partial(
      pl.kernel,
      out_type=(out_sd, out_sd),
      mesh=mesh,
      scratch_types=[
          pltpu.VMEM((nchunks, CHUNK), jnp.int32),
          pltpu.VMEM((nchunks, CHUNK), jnp.int32),
          pltpu.VMEM((CHUNK, d), jnp.float32),
          pltpu.VMEM((LANES,), jnp.float32),
          pltpu.VMEM((LANES,), jnp.float32),
          pltpu.SemaphoreType.DMA,
      ],
  )
  def sc_lookup(enc_idx, dec_idx, enc_scale, dec_scale, table,
                enc_out, dec_out,
                enc_idx_v, dec_idx_v, buf, enc_sc_v, dec_sc_v, sem):
    wid = lax.axis_index("s") * info.num_cores + lax.axis_index("c")
    base = wid * per_w

    pltpu.sync_copy(enc_idx.at[wid], enc_idx_v)
    pltpu.sync_copy(dec_idx.at[wid], dec_idx_v)
    pltpu.sync_copy(enc_scale, enc_sc_v)
    pltpu.sync_copy(dec_scale, dec_sc_v)

    for idx_v, sc_v, out in ((enc_idx_v, enc_sc_v, enc_out),
                             (dec_idx_v, dec_sc_v, dec_out)):
      sc = sc_v[...]

      @pl.loop(0, nchunks)
      def _chunk(j):
        pltpu.async_copy(table.at[idx_v.at[j]], buf, sem).wait()

        @pl.loop(0, CHUNK)
        def _row(i):
          for k in range(d // LANES):
            buf[i, pl.ds(k * LANES, LANES)] = buf[i, pl.ds(k * LANES, LANES)] * sc

        pltpu.sync_copy(buf, out.at[pl.ds(base + j * CHUNK, CHUNK)])

  return sc_lookup, nw, nchunks


def kernel(input_ids, encoder_embed_scale, decoder_input_ids,
           decoder_embed_scale, shared_weight):
  b, l = input_ids.shape
  vocab, d = shared_weight.shape
  n_rows = b * l

  sc_lookup, nw, nchunks = _make_sc_lookup(vocab, d, n_rows)

  enc_idx = input_ids.astype(jnp.int32).reshape(nw, nchunks, CHUNK)
  dec_idx = decoder_input_ids.astype(jnp.int32).reshape(nw, nchunks, CHUNK)
  enc_s = jnp.broadcast_to(encoder_embed_scale.astype(jnp.float32), (LANES,))
  dec_s = jnp.broadcast_to(decoder_embed_scale.astype(jnp.float32), (LANES,))

  enc_out, dec_out = sc_lookup(enc_idx, dec_idx, enc_s, dec_s, shared_weight)
  return (enc_out.reshape(b, l, d), dec_out.reshape(b, l, d))


# speedup vs baseline: 4.9397x; 1.0061x over previous
import functools

import jax
import jax.numpy as jnp
from jax import lax
from jax.experimental import pallas as pl
from jax.experimental.pallas import tpu as pltpu
from jax.experimental.pallas import tpu_sc as plsc

LANES = 16
CHUNK = 128
NBUF = 3


def _make_sc_lookup(vocab, d, n_rows):
  info = plsc.get_sparse_core_info()
  nw = info.num_cores * info.num_subcores
  assert n_rows % (nw * CHUNK) == 0
  per_w = n_rows // nw
  ncs = per_w // CHUNK
  nch = 2 * ncs
  assert nch >= 7 and (nch - 4) % NBUF == 0

  mesh = plsc.VectorSubcoreMesh(core_axis_name="c", subcore_axis_name="s")
  out_sd = jax.ShapeDtypeStruct((2 * n_rows, d), jnp.float32)

  @functools.partial(
      pl.kernel,
      out_type=out_sd,
      mesh=mesh,
      scratch_types=[
          pltpu.VMEM((nch, CHUNK), jnp.int32),
          pltpu.VMEM((CHUNK, d), jnp.float32),
          pltpu.VMEM((CHUNK, d), jnp.float32),
          pltpu.VMEM((CHUNK, d), jnp.float32),
          pltpu.VMEM((LANES,), jnp.float32),
          pltpu.VMEM((LANES,), jnp.float32),
          pltpu.SemaphoreType.DMA,
          pltpu.SemaphoreType.DMA,
          pltpu.SemaphoreType.DMA,
          pltpu.SemaphoreType.DMA,
          pltpu.SemaphoreType.DMA,
          pltpu.SemaphoreType.DMA,
      ],
  )
  def sc_lookup(idx_all, enc_scale, dec_scale, table, out,
                idx_v, buf0, buf1, buf2, enc_sc_v, dec_sc_v,
                g0, g1, g2, o0, o1, o2):
    bufs = (buf0, buf1, buf2)
    gsems = (g0, g1, g2)
    osems = (o0, o1, o2)

    wid = lax.axis_index("s") * info.num_cores + lax.axis_index("c")
    base = wid * per_w

    pltpu.sync_copy(idx_all.at[wid], idx_v)
    pltpu.sync_copy(enc_scale, enc_sc_v)
    pltpu.sync_copy(dec_scale, dec_sc_v)
    enc_sc = enc_sc_v[...]
    dec_sc = dec_sc_v[...]

    def off(c):
      return base + c * CHUNK + jnp.where(c < ncs, 0, n_rows - per_w)

    def start_gather(c, b):
      pltpu.async_copy(table.at[idx_v.at[c]], bufs[b], gsems[b])

    def wait_gather(c, b):
      pltpu.make_async_copy(table.at[idx_v.at[c]], bufs[b], gsems[b]).wait()

    def start_scatter(c, b):
      pltpu.async_copy(bufs[b], out.at[pl.ds(off(c), CHUNK)], osems[b])

    def wait_scatter(c, b):
      pltpu.make_async_copy(bufs[b], out.at[pl.ds(off(c), CHUNK)],
                            osems[b]).wait()

    def scale(c, b):
      buf = bufs[b]
      sc = jnp.where(c < ncs, enc_sc, dec_sc)

      @pl.loop(0, CHUNK)
      def _row(i):
        for k in range(d // LANES):
          buf[i, pl.ds(k * LANES, LANES)] = buf[i, pl.ds(k * LANES, LANES)] * sc

    start_gather(0, 0)
    start_gather(1, 1)
    wait_gather(0, 0)
    scale(0, 0)
    start_scatter(0, 0)
    start_gather(2, 2)
    wait_gather(1, 1)
    scale(1, 1)
    start_scatter(1, 1)
    wait_scatter(0, 0)
    start_gather(3, 0)

    @pl.loop(2, nch - 2, step=NBUF)
    def _main(j):
      for t in range(NBUF):
        c = j + t
        b = (2 + t) % NBUF
        wait_gather(c, b)
        scale(c, b)
        start_scatter(c, b)
        wait_scatter(c - 1, (b + 2) % NBUF)
        start_gather(c + 2, (b + 2) % NBUF)

    for c in (nch - 2, nch - 1):
      b = c % NBUF
      wait_gather(c, b)
      scale(c, b)
      start_scatter(c, b)
      wait_scatter(c - 1, (c - 1) % NBUF)
    wait_scatter(nch - 1, (nch - 1) % NBUF)

  return sc_lookup, nw, ncs


def kernel(input_ids, encoder_embed_scale, decoder_input_ids,
           decoder_embed_scale, shared_weight):
  b, l = input_ids.shape
  vocab, d = shared_weight.shape
  n_rows = b * l

  sc_lookup, nw, ncs = _make_sc_lookup(vocab, d, n_rows)

  enc_idx = input_ids.astype(jnp.int32).reshape(nw, ncs, CHUNK)
  dec_idx = decoder_input_ids.astype(jnp.int32).reshape(nw, ncs, CHUNK)
  idx_all = jnp.concatenate([enc_idx, dec_idx], axis=1)
  enc_s = jnp.broadcast_to(encoder_embed_scale.astype(jnp.float32), (LANES,))
  dec_s = jnp.broadcast_to(decoder_embed_scale.astype(jnp.float32), (LANES,))

  out = sc_lookup(idx_all, enc_s, dec_s, shared_weight)
  enc_out = out[:n_rows].reshape(b, l, d)
  dec_out = out[n_rows:].reshape(b, l, d)
  return (enc_out, dec_out)


# speedup vs baseline: 8.4693x; 1.7146x over previous
import functools

import jax
import jax.numpy as jnp
from jax import lax
from jax.experimental import pallas as pl
from jax.experimental.pallas import tpu as pltpu
from jax.experimental.pallas import tpu_sc as plsc

LANES = 16
CHUNK = 128
NBUF = 3


def _make_sc_lookup(vocab, d, n_rows):
  info = plsc.get_sparse_core_info()
  nw = info.num_cores * info.num_subcores
  assert n_rows % (nw * CHUNK) == 0
  per_w = n_rows // nw
  ncs = per_w // CHUNK
  assert ncs >= 5
  n_main = (ncs - 2 - NBUF) // NBUF * NBUF
  tail_lo = 2 + n_main

  mesh = plsc.VectorSubcoreMesh(core_axis_name="c", subcore_axis_name="s")
  out_sd = jax.ShapeDtypeStruct((n_rows, d), jnp.float32)

  @functools.partial(
      pl.kernel,
      out_type=(out_sd, out_sd),
      mesh=mesh,
      scratch_types=[
          pltpu.VMEM((ncs, CHUNK), jnp.int32),
          pltpu.VMEM((ncs, CHUNK), jnp.int32),
          pltpu.VMEM((CHUNK, d), jnp.float32),
          pltpu.VMEM((CHUNK, d), jnp.float32),
          pltpu.VMEM((CHUNK, d), jnp.float32),
          pltpu.VMEM((LANES,), jnp.float32),
          pltpu.VMEM((LANES,), jnp.float32),
          pltpu.SemaphoreType.DMA,
          pltpu.SemaphoreType.DMA,
          pltpu.SemaphoreType.DMA,
          pltpu.SemaphoreType.DMA,
          pltpu.SemaphoreType.DMA,
          pltpu.SemaphoreType.DMA,
      ],
  )
  def sc_lookup(enc_idx, dec_idx, enc_scale, dec_scale, table,
                enc_out, dec_out,
                enc_idx_v, dec_idx_v, buf0, buf1, buf2, enc_sc_v, dec_sc_v,
                g0, g1, g2, o0, o1, o2):
    bufs = (buf0, buf1, buf2)
    gsems = (g0, g1, g2)
    osems = (o0, o1, o2)

    wid = lax.axis_index("s") * info.num_cores + lax.axis_index("c")
    base = wid * per_w

    pltpu.sync_copy(enc_idx.at[wid], enc_idx_v)
    pltpu.sync_copy(dec_idx.at[wid], dec_idx_v)
    pltpu.sync_copy(enc_scale, enc_sc_v)
    pltpu.sync_copy(dec_scale, dec_sc_v)

    def run_stream(idx_v, sc_v, out):
      sc = sc_v[...]

      def start_gather(c, b):
        pltpu.async_copy(table.at[idx_v.at[c]], bufs[b], gsems[b])

      def wait_gather(c, b):
        pltpu.make_async_copy(table.at[idx_v.at[c]], bufs[b], gsems[b]).wait()

      def start_scatter(c, b):
        pltpu.async_copy(bufs[b], out.at[pl.ds(base + c * CHUNK, CHUNK)],
                         osems[b])

      def wait_scatter(c, b):
        pltpu.make_async_copy(bufs[b], out.at[pl.ds(base + c * CHUNK, CHUNK)],
                              osems[b]).wait()

      def scale(b):
        buf = bufs[b]

        @pl.loop(0, CHUNK)
        def _row(i):
          for k in range(d // LANES):
            buf[i, pl.ds(k * LANES, LANES)] = (
                buf[i, pl.ds(k * LANES, LANES)] * sc)

      start_gather(0, 0)
      start_gather(1, 1)
      wait_gather(0, 0)
      scale(0)
      start_scatter(0, 0)
      start_gather(2, 2)
      wait_gather(1, 1)
      scale(1)
      start_scatter(1, 1)
      wait_scatter(0, 0)
      start_gather(3, 0)

      @pl.loop(2, tail_lo, step=NBUF)
      def _main(j):
        for t in range(NBUF):
          c = j + t
          b = (2 + t) % NBUF
          wait_gather(c, b)
          scale(b)
          start_scatter(c, b)
          wait_scatter(c - 1, (b + 2) % NBUF)
          start_gather(c + 2, (b + 2) % NBUF)

      for c in range(tail_lo, ncs):
        b = c % NBUF
        wait_gather(c, b)
        scale(b)
        start_scatter(c, b)
        wait_scatter(c - 1, (c - 1) % NBUF)
        if c + 2 < ncs:
          start_gather(c + 2, (c + 2) % NBUF)
      wait_scatter(ncs - 1, (ncs - 1) % NBUF)

    run_stream(enc_idx_v, enc_sc_v, enc_out)
    run_stream(dec_idx_v, dec_sc_v, dec_out)

  return sc_lookup, nw, ncs


def kernel(input_ids, encoder_embed_scale, decoder_input_ids,
           decoder_embed_scale, shared_weight):
  b, l = input_ids.shape
  vocab, d = shared_weight.shape
  n_rows = b * l

  sc_lookup, nw, ncs = _make_sc_lookup(vocab, d, n_rows)

  enc_idx = input_ids.astype(jnp.int32).reshape(nw, ncs, CHUNK)
  dec_idx = decoder_input_ids.astype(jnp.int32).reshape(nw, ncs, CHUNK)
  enc_s = jnp.broadcast_to(encoder_embed_scale.astype(jnp.float32), (LANES,))
  dec_s = jnp.broadcast_to(decoder_embed_scale.astype(jnp.float32), (LANES,))

  enc_out, dec_out = sc_lookup(enc_idx, dec_idx, enc_s, dec_s, shared_weight)
  return (enc_out.reshape(b, l, d), dec_out.reshape(b, l, d))
